# baseline (speedup 1.0000x reference)
import functools

import jax
import jax.numpy as jnp
from jax import lax
from jax.experimental import pallas as pl
from jax.experimental.pallas import tpu as pltpu
from jax.experimental.pallas import tpu_sc as plsc

B, N, S, K, C = 16, 4096, 512, 32, 64
QC = 128
CH = 256
NCH = N // CH
F32 = jnp.float32
I32 = jnp.int32

NC_SC, NS_SC = 2, 16
NW = NC_SC * NS_SC
ROWS = B * K * S
ROWS_W = ROWS // NW
CHUNK = 128
NCHUNK = ROWS_W // CHUNK

_HI = jax.lax.Precision.HIGHEST


def _fps_body(xt_ref, out_ref, dist_ref, far_ref):
    i = pl.program_id(0)

    @pl.when(i == 0)
    def _init():
        dist_ref[...] = jnp.full((B, N), 1e10, F32)
        far_ref[...] = jnp.zeros((B, 128), I32)

    x = xt_ref[0]
    y = xt_ref[1]
    z = xt_ref[2]
    far = far_ref[:, 0:1]
    lane = lax.broadcasted_iota(I32, (B, N), 1)
    oh = lane == far
    ninf = jnp.float32(-jnp.inf)
    cx = jnp.max(jnp.where(oh, x, ninf), axis=1, keepdims=True)
    cy = jnp.max(jnp.where(oh, y, ninf), axis=1, keepdims=True)
    cz = jnp.max(jnp.where(oh, z, ninf), axis=1, keepdims=True)
    out_ref[...] = jnp.concatenate([cx, cy, cz], axis=1)[None]
    dx = x - cx
    dy = y - cy
    dz = z - cz
    d = dx * dx + dy * dy + dz * dz
    dist = dist_ref[...]
    dist = jnp.where(d < dist, d, dist)
    dist_ref[...] = dist
    m = jnp.max(dist, axis=1, keepdims=True)
    nxt = jnp.min(jnp.where(dist == m, lane, jnp.int32(N)),
                  axis=1, keepdims=True)
    far_ref[...] = jnp.broadcast_to(nxt, (B, 128))


def _fps(xt):
    return pl.pallas_call(
        _fps_body,
        grid=(S,),
        in_specs=[pl.BlockSpec((3, B, N), lambda i: (0, 0, 0))],
        out_specs=pl.BlockSpec((1, B, 3), lambda i: (i, 0, 0)),
        out_shape=jax.ShapeDtypeStruct((S, B, 3), F32),
        scratch_shapes=[pltpu.VMEM((B, N), F32), pltpu.VMEM((B, 128), I32)],
    )(xt)


def _g0_body(xyz_ref, pts_ref, w0t_ref, g0_ref):
    xyz = xyz_ref[0]
    pts = pts_ref[0]
    w = w0t_ref[...]
    g = (jnp.dot(xyz, w[0:3], precision=_HI, preferred_element_type=F32)
         + jnp.dot(pts, w[3:], precision=_HI, preferred_element_type=F32))
    g0_ref[...] = g[None]


def _g0(xyz, points, w0t):
    return pl.pallas_call(
        _g0_body,
        grid=(B,),
        in_specs=[
            pl.BlockSpec((1, N, 3), lambda b: (b, 0, 0)),
            pl.BlockSpec((1, N, C), lambda b: (b, 0, 0)),
            pl.BlockSpec((C + 3, C), lambda b: (0, 0)),
        ],
        out_specs=pl.BlockSpec((1, N, C), lambda b: (b, 0, 0)),
        out_shape=jax.ShapeDtypeStruct((B, N, C), F32),
    )(xyz, points, w0t)


def _sel_body(xyz_ref, nxt_ref, idx_ref, scr_ref):
    b = pl.program_id(0)
    q3 = nxt_ref[0]
    xq = q3[0:1, :]
    yq = q3[1:2, :]
    zq = q3[2:3, :]
    qsum = xq * xq + yq * yq + zq * zq
    q3b = q3.astype(jnp.bfloat16)
    for c in range(NCH):
        p = xyz_ref[0, pl.ds(c * CH, CH), :]
        psum = (p[:, 0:1] * p[:, 0:1] + p[:, 1:2] * p[:, 1:2]
                + p[:, 2:3] * p[:, 2:3])
        mm = jnp.dot(p.astype(jnp.bfloat16), q3b,
                     preferred_element_type=F32)
        d = -2.0 * mm
        d = d + qsum
        d = d + psum
        bits = lax.bitcast_convert_type(d, I32)
        sub = lax.broadcasted_iota(I32, (CH, QC), 0) + c * CH
        scr_ref[pl.ds(c * CH, CH), :] = (bits & jnp.int32(-4096)) | sub

    def _stage(v, d, size):
        m = v.shape[0]
        g = m // (2 * d)
        r = v.reshape(g, 2, d, QC)
        lo, hi = r[:, 0], r[:, 1]
        mn = jnp.minimum(lo, hi)
        mx = jnp.maximum(lo, hi)
        asc = (lax.broadcasted_iota(I32, (g, 1, 1), 0) * (2 * d) & size) == 0
        return jnp.concatenate(
            [jnp.where(asc, mn, mx)[:, None], jnp.where(asc, mx, mn)[:, None]],
            axis=1).reshape(m, QC)

    def _sortblocks(g, carry):
        v = scr_ref[pl.ds(g * 64, 64), :]
        for size in (2, 4, 8, 16, 32):
            d = size // 2
            while d:
                v = _stage(v, d, size)
                d //= 2
        scr_ref[pl.ds(g * 64, 64), :] = v
        return carry

    lax.fori_loop(0, N // 64, _sortblocks, 0)

    def _merge_round(npairs):
        def body(g, carry):
            a = scr_ref[pl.ds(g * 64, 32), :]
            bb = scr_ref[pl.ds(g * 64 + 32, 32), :]
            v = jnp.minimum(a, bb)
            asc = (g & 1) == 0
            for d in (16, 8, 4, 2, 1):
                gg = 32 // (2 * d)
                r = v.reshape(gg, 2, d, QC)
                mn = jnp.minimum(r[:, 0], r[:, 1])
                mx = jnp.maximum(r[:, 0], r[:, 1])
                v = jnp.concatenate(
                    [jnp.where(asc, mn, mx)[:, None],
                     jnp.where(asc, mx, mn)[:, None]], axis=1).reshape(32, QC)
            scr_ref[pl.ds(g * 32, 32), :] = v
            return carry

        lax.fori_loop(0, npairs, body, 0)

    npairs = N // 64
    while npairs >= 1:
        _merge_round(npairs)
        npairs //= 2

    idx_ref[0] = (scr_ref[0:K, :] & 4095) + b * N


def _select(xyz, nxt):
    return pl.pallas_call(
        _sel_body,
        grid=(B, S // QC),
        in_specs=[
            pl.BlockSpec((1, N, 3), lambda b, q: (b, 0, 0)),
            pl.BlockSpec((1, 3, QC), lambda b, q: (b, 0, q)),
        ],
        out_specs=pl.BlockSpec((1, K, QC), lambda b, q: (b, 0, q)),
        out_shape=jax.ShapeDtypeStruct((B, K, S), I32),
        scratch_shapes=[pltpu.VMEM((N, QC), I32)],
    )(xyz, nxt)


def _sc_gather(table, idx3):
    mesh = plsc.VectorSubcoreMesh(core_axis_name="c", subcore_axis_name="s")

    @functools.partial(
        pl.kernel,
        mesh=mesh,
        compiler_params=pltpu.CompilerParams(use_tc_tiling_on_sc=False),
        out_type=jax.ShapeDtypeStruct((ROWS, C), F32),
        scratch_types=[
            pltpu.VMEM((NCHUNK, CHUNK), I32),
            pltpu.VMEM((CHUNK, C), F32),
            pltpu.SemaphoreType.DMA,
        ],
    )
    def k(table_hbm, idx_hbm, out_hbm, idx_v, buf, sem):
        wid = lax.axis_index("s") * NC_SC + lax.axis_index("c")
        base = wid * ROWS_W
        pltpu.sync_copy(idx_hbm.at[wid], idx_v)

        def body(c, carry):
            pltpu.async_copy(table_hbm.at[idx_v.at[c]], buf, sem).wait()
            pltpu.sync_copy(buf, out_hbm.at[pl.ds(base + c * CHUNK, CHUNK)])
            return carry

        lax.fori_loop(0, NCHUNK, body, 0)

    return k(table, idx3)


def _mlp_body(g_ref, nxy_ref, w0t_ref, b0_ref, w1t_ref, b1_ref,
              w2t_ref, b2_ref, out_ref):
    nxy = nxy_ref[0]
    q0 = (b0_ref[...][None, :]
          - jnp.dot(nxy, w0t_ref[0:3], precision=_HI,
                    preferred_element_type=F32))
    b1 = b1_ref[...][None, :]
    b2 = b2_ref[...][None, :]
    w1 = w1t_ref[...]
    w2 = w2t_ref[...]
    a = jnp.maximum(g_ref[0] + q0[None], 0.0).reshape(K * S, C)
    h = jnp.maximum(
        jnp.dot(a, w1, precision=_HI, preferred_element_type=F32) + b1, 0.0)
    o = jnp.maximum(
        jnp.dot(h, w2, precision=_HI, preferred_element_type=F32) + b2, 0.0)
    out_ref[...] = jnp.max(o.reshape(K, S, 2 * C), axis=0)[None]


def _mlp(g, new_xyz, w0t, b0, w1t, b1, w2t, b2):
    return pl.pallas_call(
        _mlp_body,
        grid=(B,),
        in_specs=[
            pl.BlockSpec((1, K, S, C), lambda b: (b, 0, 0, 0)),
            pl.BlockSpec((1, S, 3), lambda b: (b, 0, 0)),
            pl.BlockSpec((C + 3, C), lambda b: (0, 0)),
            pl.BlockSpec((C,), lambda b: (0,)),
            pl.BlockSpec((C, C), lambda b: (0, 0)),
            pl.BlockSpec((C,), lambda b: (0,)),
            pl.BlockSpec((C, 2 * C), lambda b: (0, 0)),
            pl.BlockSpec((2 * C,), lambda b: (0,)),
        ],
        out_specs=pl.BlockSpec((1, S, 2 * C), lambda b: (b, 0, 0)),
        out_shape=jax.ShapeDtypeStruct((B, S, 2 * C), F32),
    )(g, new_xyz, w0t, b0, w1t, b1, w2t, b2)


def kernel(xyz, points, W0, b0, W1, b1, W2, b2):
    xt = jnp.transpose(xyz, (2, 0, 1))
    nxt = _fps(xt)
    new_xyz = jnp.transpose(nxt, (1, 0, 2))
    w0t = W0.T
    w1t = W1.T
    w2t = W2.T
    g0 = _g0(xyz, points, w0t)
    idx = _select(xyz, jnp.transpose(nxt, (1, 2, 0)))
    g = _sc_gather(g0.reshape(B * N, C),
                   idx.reshape(NW, NCHUNK, CHUNK))
    out = _mlp(g.reshape(B, K, S, C), new_xyz,
               w0t, b0, w1t, b1, w2t, b2)
    return (new_xyz, out)


# speedup vs baseline: 2.0091x; 2.0091x over previous
import functools

import jax
import jax.numpy as jnp
from jax import lax
from jax.experimental import pallas as pl
from jax.experimental.pallas import tpu as pltpu
from jax.experimental.pallas import tpu_sc as plsc

B, N, S, K, C = 16, 4096, 512, 32, 64
QC = 128
CH = 256
NCH = N // CH
F32 = jnp.float32
I32 = jnp.int32

NC_SC, NS_SC = 2, 16
NW = NC_SC * NS_SC
ROWS = B * K * S
ROWS_W = ROWS // NW
CHUNK = 128
NCHUNK = ROWS_W // CHUNK

_HI = jax.lax.Precision.HIGHEST


def _fps_body(xt_ref, out_ref, dist_ref, far_ref):
    i = pl.program_id(0)

    @pl.when(i == 0)
    def _init():
        dist_ref[...] = jnp.full((B, N), 1e10, F32)
        far_ref[...] = jnp.zeros((B, 128), I32)

    x = xt_ref[0]
    y = xt_ref[1]
    z = xt_ref[2]
    far = far_ref[:, 0:1]
    lane = lax.broadcasted_iota(I32, (B, N), 1)
    oh = lane == far
    ninf = jnp.float32(-jnp.inf)
    cx = jnp.max(jnp.where(oh, x, ninf), axis=1, keepdims=True)
    cy = jnp.max(jnp.where(oh, y, ninf), axis=1, keepdims=True)
    cz = jnp.max(jnp.where(oh, z, ninf), axis=1, keepdims=True)
    out_ref[...] = jnp.concatenate([cx, cy, cz], axis=1)[None]
    dx = x - cx
    dy = y - cy
    dz = z - cz
    d = dx * dx + dy * dy + dz * dz
    dist = dist_ref[...]
    dist = jnp.where(d < dist, d, dist)
    dist_ref[...] = dist
    m = jnp.max(dist, axis=1, keepdims=True)
    nxt = jnp.min(jnp.where(dist == m, lane, jnp.int32(N)),
                  axis=1, keepdims=True)
    far_ref[...] = jnp.broadcast_to(nxt, (B, 128))


def _fps(xt):
    return pl.pallas_call(
        _fps_body,
        grid=(S,),
        in_specs=[pl.BlockSpec((3, B, N), lambda i: (0, 0, 0))],
        out_specs=pl.BlockSpec((1, B, 3), lambda i: (i, 0, 0)),
        out_shape=jax.ShapeDtypeStruct((S, B, 3), F32),
        scratch_shapes=[pltpu.VMEM((B, N), F32), pltpu.VMEM((B, 128), I32)],
    )(xt)


def _g0_body(xyz_ref, pts_ref, w0t_ref, g0_ref):
    xyz = xyz_ref[0]
    pts = pts_ref[0]
    w = w0t_ref[...]
    g = (jnp.dot(xyz, w[0:3], precision=_HI, preferred_element_type=F32)
         + jnp.dot(pts, w[3:], precision=_HI, preferred_element_type=F32))
    g0_ref[...] = g[None]


def _g0(xyz, points, w0t):
    return pl.pallas_call(
        _g0_body,
        grid=(B,),
        in_specs=[
            pl.BlockSpec((1, N, 3), lambda b: (b, 0, 0)),
            pl.BlockSpec((1, N, C), lambda b: (b, 0, 0)),
            pl.BlockSpec((C + 3, C), lambda b: (0, 0)),
        ],
        out_specs=pl.BlockSpec((1, N, C), lambda b: (b, 0, 0)),
        out_shape=jax.ShapeDtypeStruct((B, N, C), F32),
    )(xyz, points, w0t)


def _sel_body(xyz_ref, nxt_ref, idx_ref, scr_ref):
    b = pl.program_id(0)
    q3 = nxt_ref[0]
    xq = q3[0:1, :]
    yq = q3[1:2, :]
    zq = q3[2:3, :]
    qsum = xq * xq + yq * yq + zq * zq
    q3b = q3.astype(jnp.bfloat16)
    for c in range(NCH):
        p = xyz_ref[0, pl.ds(c * CH, CH), :]
        psum = (p[:, 0:1] * p[:, 0:1] + p[:, 1:2] * p[:, 1:2]
                + p[:, 2:3] * p[:, 2:3])
        mm = jnp.dot(p.astype(jnp.bfloat16), q3b,
                     preferred_element_type=F32)
        d = -2.0 * mm
        d = d + qsum
        d = d + psum
        bits = lax.bitcast_convert_type(d, I32)
        sub = lax.broadcasted_iota(I32, (CH, QC), 0) + c * CH
        packed = (bits & jnp.int32(-4096)) | sub
        scr_ref[pl.ds(c * (CH // 128), CH // 128), :, :] = (
            packed.reshape(CH // 128, 128, QC))

    def _ce(v, d, asc):
        g = 32 // (2 * d)
        r = v.reshape((g, 2, d) + v.shape[1:])
        lo, hi = r[:, 0], r[:, 1]
        mn = jnp.minimum(lo, hi)
        mx = jnp.maximum(lo, hi)
        if isinstance(asc, bool):
            a, bb = (mn, mx) if asc else (mx, mn)
        else:
            a = jnp.where(asc, mn, mx)
            bb = jnp.where(asc, mx, mn)
        return jnp.concatenate([a[:, None], bb[:, None]], axis=1).reshape(v.shape)

    def _ce_pat(v, d, size):
        g = 32 // (2 * d)
        r = v.reshape((g, 2, d) + v.shape[1:])
        lo, hi = r[:, 0], r[:, 1]
        mn = jnp.minimum(lo, hi)
        mx = jnp.maximum(lo, hi)
        outs = []
        for gi in range(g):
            if ((gi * 2 * d) & size) == 0:
                outs.extend([mn[gi], mx[gi]])
            else:
                outs.extend([mx[gi], mn[gi]])
        return jnp.concatenate(outs, axis=0).reshape(v.shape)

    for j0 in range(0, 128, 8):
        v = scr_ref[:, pl.ds(j0, 8), :]
        for size in (2, 4, 8, 16):
            d = size // 2
            while d:
                v = _ce_pat(v, d, size)
                d //= 2
        asc = (j0 & 64) == 0
        for d in (16, 8, 4, 2, 1):
            v = _ce(v, d, asc)
        scr_ref[:, pl.ds(j0, 8), :] = v

    for h in (64, 32, 16, 8):
        step = min(8, h)
        for j0 in range(0, h, step):
            a = scr_ref[:, pl.ds(j0, step), :]
            bb = scr_ref[:, pl.ds(j0 + h, step), :]
            v = jnp.minimum(a, bb)
            if h // 2 >= 8:
                asc = (j0 & (h // 2)) == 0
            else:
                asc = (lax.broadcasted_iota(I32, (1, step, 1), 1)
                       & (h // 2)) == 0
            for d in (16, 8, 4, 2, 1):
                v = _ce(v, d, asc)
            scr_ref[:, pl.ds(j0, step), :] = v

    v = scr_ref[:, 0:8, :]
    for h in (4, 2, 1):
        c = jnp.minimum(v[:, :h], v[:, h:2 * h])
        if h > 1:
            asc = (lax.broadcasted_iota(I32, (1, h, 1), 1) & (h // 2)) == 0
        else:
            asc = True
        for d in (16, 8, 4, 2, 1):
            c = _ce(c, d, asc)
        v = c

    idx_ref[0] = (v[:, 0, :] & 4095) + b * N


def _select(xyz, nxt):
    return pl.pallas_call(
        _sel_body,
        grid=(B, S // QC),
        in_specs=[
            pl.BlockSpec((1, N, 3), lambda b, q: (b, 0, 0)),
            pl.BlockSpec((1, 3, QC), lambda b, q: (b, 0, q)),
        ],
        out_specs=pl.BlockSpec((1, K, QC), lambda b, q: (b, 0, q)),
        out_shape=jax.ShapeDtypeStruct((B, K, S), I32),
        scratch_shapes=[pltpu.VMEM((K, N // K, QC), I32)],
    )(xyz, nxt)


def _sc_gather(table, idx3):
    mesh = plsc.VectorSubcoreMesh(core_axis_name="c", subcore_axis_name="s")

    @functools.partial(
        pl.kernel,
        mesh=mesh,
        compiler_params=pltpu.CompilerParams(use_tc_tiling_on_sc=False),
        out_type=jax.ShapeDtypeStruct((ROWS, C), F32),
        scratch_types=[
            pltpu.VMEM((NCHUNK, CHUNK), I32),
            pltpu.VMEM((CHUNK, C), F32),
            pltpu.SemaphoreType.DMA,
        ],
    )
    def k(table_hbm, idx_hbm, out_hbm, idx_v, buf, sem):
        wid = lax.axis_index("s") * NC_SC + lax.axis_index("c")
        base = wid * ROWS_W
        pltpu.sync_copy(idx_hbm.at[wid], idx_v)

        def body(c, carry):
            pltpu.async_copy(table_hbm.at[idx_v.at[c]], buf, sem).wait()
            pltpu.sync_copy(buf, out_hbm.at[pl.ds(base + c * CHUNK, CHUNK)])
            return carry

        lax.fori_loop(0, NCHUNK, body, 0)

    return k(table, idx3)


def _mlp_body(g_ref, nxy_ref, w0t_ref, b0_ref, w1t_ref, b1_ref,
              w2t_ref, b2_ref, out_ref):
    nxy = nxy_ref[0]
    q0 = (b0_ref[...][None, :]
          - jnp.dot(nxy, w0t_ref[0:3], precision=_HI,
                    preferred_element_type=F32))
    b1 = b1_ref[...][None, :]
    b2 = b2_ref[...][None, :]
    w1 = w1t_ref[...]
    w2 = w2t_ref[...]
    a = jnp.maximum(g_ref[0] + q0[None], 0.0).reshape(K * S, C)
    h = jnp.maximum(
        jnp.dot(a, w1, precision=_HI, preferred_element_type=F32) + b1, 0.0)
    o = jnp.maximum(
        jnp.dot(h, w2, precision=_HI, preferred_element_type=F32) + b2, 0.0)
    out_ref[...] = jnp.max(o.reshape(K, S, 2 * C), axis=0)[None]


def _mlp(g, new_xyz, w0t, b0, w1t, b1, w2t, b2):
    return pl.pallas_call(
        _mlp_body,
        grid=(B,),
        in_specs=[
            pl.BlockSpec((1, K, S, C), lambda b: (b, 0, 0, 0)),
            pl.BlockSpec((1, S, 3), lambda b: (b, 0, 0)),
            pl.BlockSpec((C + 3, C), lambda b: (0, 0)),
            pl.BlockSpec((C,), lambda b: (0,)),
            pl.BlockSpec((C, C), lambda b: (0, 0)),
            pl.BlockSpec((C,), lambda b: (0,)),
            pl.BlockSpec((C, 2 * C), lambda b: (0, 0)),
            pl.BlockSpec((2 * C,), lambda b: (0,)),
        ],
        out_specs=pl.BlockSpec((1, S, 2 * C), lambda b: (b, 0, 0)),
        out_shape=jax.ShapeDtypeStruct((B, S, 2 * C), F32),
    )(g, new_xyz, w0t, b0, w1t, b1, w2t, b2)


def kernel(xyz, points, W0, b0, W1, b1, W2, b2):
    xt = jnp.transpose(xyz, (2, 0, 1))
    nxt = _fps(xt)
    new_xyz = jnp.transpose(nxt, (1, 0, 2))
    w0t = W0.T
    w1t = W1.T
    w2t = W2.T
    g0 = _g0(xyz, points, w0t)
    idx = _select(xyz, jnp.transpose(nxt, (1, 2, 0)))
    g = _sc_gather(g0.reshape(B * N, C),
                   idx.reshape(NW, NCHUNK, CHUNK))
    out = _mlp(g.reshape(B, K, S, C), new_xyz,
               w0t, b0, w1t, b1, w2t, b2)
    return (new_xyz, out)


# speedup vs baseline: 2.0478x; 1.0193x over previous
import functools

import jax
import jax.numpy as jnp
from jax import lax
from jax.experimental import pallas as pl
from jax.experimental.pallas import tpu as pltpu
from jax.experimental.pallas import tpu_sc as plsc

B, N, S, K, C = 16, 4096, 512, 32, 64
QC = 128
CH = 256
NCH = N // CH
F32 = jnp.float32
I32 = jnp.int32

NC_SC, NS_SC = 2, 16
NW = NC_SC * NS_SC
ROWS = B * K * S
ROWS_W = ROWS // NW
CHUNK = 128
NCHUNK = ROWS_W // CHUNK

_HI = jax.lax.Precision.HIGHEST


def _fps_body(xt_ref, out_ref, dist_ref, far_ref):
    i = pl.program_id(0)

    @pl.when(i == 0)
    def _init():
        dist_ref[...] = jnp.full((B, N), 1e10, F32)
        far_ref[...] = jnp.zeros((B, 128), I32)

    x = xt_ref[0]
    y = xt_ref[1]
    z = xt_ref[2]
    far = far_ref[:, 0:1]
    lane = lax.broadcasted_iota(I32, (B, N), 1)
    oh = lane == far
    ninf = jnp.float32(-jnp.inf)
    dist0 = dist_ref[...]
    stack = jnp.concatenate(
        [jnp.where(oh, x, ninf), jnp.where(oh, y, ninf),
         jnp.where(oh, z, ninf)], axis=0)
    red = jnp.max(stack, axis=1, keepdims=True)
    cx = red[0:B]
    cy = red[B:2 * B]
    cz = red[2 * B:]
    out_ref[...] = jnp.concatenate([cx, cy, cz], axis=1)[None]
    dx = x - cx
    dy = y - cy
    dz = z - cz
    d = dx * dx + dy * dy + dz * dz
    dist = jnp.where(d < dist0, d, dist0)
    dist_ref[...] = dist
    m = jnp.max(dist, axis=1, keepdims=True)
    nxt = jnp.min(jnp.where(dist == m, lane, jnp.int32(N)),
                  axis=1, keepdims=True)
    far_ref[...] = jnp.broadcast_to(nxt, (B, 128))


def _fps(xt):
    return pl.pallas_call(
        _fps_body,
        grid=(S,),
        in_specs=[pl.BlockSpec((3, B, N), lambda i: (0, 0, 0))],
        out_specs=pl.BlockSpec((1, B, 3), lambda i: (i, 0, 0)),
        out_shape=jax.ShapeDtypeStruct((S, B, 3), F32),
        scratch_shapes=[pltpu.VMEM((B, N), F32), pltpu.VMEM((B, 128), I32)],
    )(xt)


def _g0_body(xyz_ref, pts_ref, w0t_ref, g0_ref):
    xyz = xyz_ref[0]
    pts = pts_ref[0]
    w = w0t_ref[...]
    g = (jnp.dot(xyz, w[0:3], precision=_HI, preferred_element_type=F32)
         + jnp.dot(pts, w[3:], precision=_HI, preferred_element_type=F32))
    g0_ref[...] = g[None]


def _g0(xyz, points, w0t):
    return pl.pallas_call(
        _g0_body,
        grid=(B,),
        in_specs=[
            pl.BlockSpec((1, N, 3), lambda b: (b, 0, 0)),
            pl.BlockSpec((1, N, C), lambda b: (b, 0, 0)),
            pl.BlockSpec((C + 3, C), lambda b: (0, 0)),
        ],
        out_specs=pl.BlockSpec((1, N, C), lambda b: (b, 0, 0)),
        out_shape=jax.ShapeDtypeStruct((B, N, C), F32),
    )(xyz, points, w0t)


def _sel_body(xyz_ref, nxt_ref, idx_ref, scr_ref):
    b = pl.program_id(0)
    q3 = nxt_ref[0]
    xq = q3[0:1, :]
    yq = q3[1:2, :]
    zq = q3[2:3, :]
    qsum = xq * xq + yq * yq + zq * zq
    q3b = q3.astype(jnp.bfloat16)
    for c in range(NCH):
        p = xyz_ref[0, pl.ds(c * CH, CH), :]
        psum = (p[:, 0:1] * p[:, 0:1] + p[:, 1:2] * p[:, 1:2]
                + p[:, 2:3] * p[:, 2:3])
        mm = jnp.dot(p.astype(jnp.bfloat16), q3b,
                     preferred_element_type=F32)
        d = -2.0 * mm
        d = d + qsum
        d = d + psum
        bits = lax.bitcast_convert_type(d, I32)
        sub = lax.broadcasted_iota(I32, (CH, QC), 0) + c * CH
        packed = (bits & jnp.int32(-4096)) | sub
        scr_ref[pl.ds(c * (CH // 128), CH // 128), :, :] = (
            packed.reshape(CH // 128, 128, QC))

    def _ce(v, d, asc):
        g = 32 // (2 * d)
        r = v.reshape((g, 2, d) + v.shape[1:])
        lo, hi = r[:, 0], r[:, 1]
        mn = jnp.minimum(lo, hi)
        mx = jnp.maximum(lo, hi)
        if isinstance(asc, bool):
            a, bb = (mn, mx) if asc else (mx, mn)
        else:
            a = jnp.where(asc, mn, mx)
            bb = jnp.where(asc, mx, mn)
        return jnp.concatenate([a[:, None], bb[:, None]], axis=1).reshape(v.shape)

    def _ce_pat(v, d, size):
        g = 32 // (2 * d)
        r = v.reshape((g, 2, d) + v.shape[1:])
        lo, hi = r[:, 0], r[:, 1]
        mn = jnp.minimum(lo, hi)
        mx = jnp.maximum(lo, hi)
        outs = []
        for gi in range(g):
            if ((gi * 2 * d) & size) == 0:
                outs.extend([mn[gi], mx[gi]])
            else:
                outs.extend([mx[gi], mn[gi]])
        return jnp.concatenate(outs, axis=0).reshape(v.shape)

    for j0 in range(0, 128, 8):
        v = scr_ref[:, pl.ds(j0, 8), :]
        for size in (2, 4, 8, 16):
            d = size // 2
            while d:
                v = _ce_pat(v, d, size)
                d //= 2
        asc = (j0 & 64) == 0
        for d in (16, 8, 4, 2, 1):
            v = _ce(v, d, asc)
        scr_ref[:, pl.ds(j0, 8), :] = v

    for h in (64, 32, 16, 8):
        step = min(8, h)
        for j0 in range(0, h, step):
            a = scr_ref[:, pl.ds(j0, step), :]
            bb = scr_ref[:, pl.ds(j0 + h, step), :]
            v = jnp.minimum(a, bb)
            if h // 2 >= 8:
                asc = (j0 & (h // 2)) == 0
            else:
                asc = (lax.broadcasted_iota(I32, (1, step, 1), 1)
                       & (h // 2)) == 0
            for d in (16, 8, 4, 2, 1):
                v = _ce(v, d, asc)
            scr_ref[:, pl.ds(j0, step), :] = v

    v = scr_ref[:, 0:8, :]
    for h in (4, 2, 1):
        c = jnp.minimum(v[:, :h], v[:, h:2 * h])
        if h > 1:
            asc = (lax.broadcasted_iota(I32, (1, h, 1), 1) & (h // 2)) == 0
        else:
            asc = True
        for d in (16, 8, 4, 2, 1):
            c = _ce(c, d, asc)
        v = c

    idx_ref[0] = (v[:, 0, :] & 4095) + b * N


def _select(xyz, nxt):
    return pl.pallas_call(
        _sel_body,
        grid=(B, S // QC),
        in_specs=[
            pl.BlockSpec((1, N, 3), lambda b, q: (b, 0, 0)),
            pl.BlockSpec((1, 3, QC), lambda b, q: (b, 0, q)),
        ],
        out_specs=pl.BlockSpec((1, K, QC), lambda b, q: (b, 0, q)),
        out_shape=jax.ShapeDtypeStruct((B, K, S), I32),
        scratch_shapes=[pltpu.VMEM((K, N // K, QC), I32)],
    )(xyz, nxt)


def _sc_gather(table, idx3):
    mesh = plsc.VectorSubcoreMesh(core_axis_name="c", subcore_axis_name="s")

    @functools.partial(
        pl.kernel,
        mesh=mesh,
        compiler_params=pltpu.CompilerParams(use_tc_tiling_on_sc=False),
        out_type=jax.ShapeDtypeStruct((ROWS, C), F32),
        scratch_types=[
            pltpu.VMEM((NCHUNK, CHUNK), I32),
            pltpu.VMEM((CHUNK, C), F32),
            pltpu.VMEM((CHUNK, C), F32),
            pltpu.SemaphoreType.DMA,
            pltpu.SemaphoreType.DMA,
        ],
    )
    def k(table_hbm, idx_hbm, out_hbm, idx_v, buf0, buf1, sem0, sem1):
        wid = lax.axis_index("s") * NC_SC + lax.axis_index("c")
        base = wid * ROWS_W
        pltpu.sync_copy(idx_hbm.at[wid], idx_v)

        def gather(c, buf, sem):
            pltpu.make_async_copy(table_hbm.at[idx_v.at[c]], buf, sem).start()

        def drain(buf, sem):
            pltpu.make_async_copy(table_hbm.at[idx_v.at[0]], buf, sem).wait()

        gather(0, buf0, sem0)
        gather(1, buf1, sem1)

        def body(c, carry):
            drain(buf0, sem0)
            pltpu.sync_copy(buf0, out_hbm.at[pl.ds(base + 2 * c * CHUNK, CHUNK)])

            @pl.when(c < NCHUNK // 2 - 1)
            def _():
                gather(2 * c + 2, buf0, sem0)

            drain(buf1, sem1)
            pltpu.sync_copy(
                buf1, out_hbm.at[pl.ds(base + (2 * c + 1) * CHUNK, CHUNK)])

            @pl.when(c < NCHUNK // 2 - 1)
            def _():
                gather(2 * c + 3, buf1, sem1)

            return carry

        lax.fori_loop(0, NCHUNK // 2, body, 0)

    return k(table, idx3)


def _mlp_body(g_ref, nxy_ref, w0t_ref, b0_ref, w1t_ref, b1_ref,
              w2t_ref, b2_ref, out_ref):
    nxy = nxy_ref[0]
    q0 = (b0_ref[...][None, :]
          - jnp.dot(nxy, w0t_ref[0:3], precision=_HI,
                    preferred_element_type=F32))
    b1 = b1_ref[...][None, :]
    b2 = b2_ref[...][None, :]
    w1 = w1t_ref[...]
    w2 = w2t_ref[...]
    a = jnp.maximum(g_ref[0] + q0[None], 0.0).reshape(K * S, C)
    h = jnp.maximum(
        jnp.dot(a, w1, precision=_HI, preferred_element_type=F32) + b1, 0.0)
    o = jnp.maximum(
        jnp.dot(h, w2, precision=_HI, preferred_element_type=F32) + b2, 0.0)
    out_ref[...] = jnp.max(o.reshape(K, S, 2 * C), axis=0)[None]


def _mlp(g, new_xyz, w0t, b0, w1t, b1, w2t, b2):
    return pl.pallas_call(
        _mlp_body,
        grid=(B,),
        in_specs=[
            pl.BlockSpec((1, K, S, C), lambda b: (b, 0, 0, 0)),
            pl.BlockSpec((1, S, 3), lambda b: (b, 0, 0)),
            pl.BlockSpec((C + 3, C), lambda b: (0, 0)),
            pl.BlockSpec((C,), lambda b: (0,)),
            pl.BlockSpec((C, C), lambda b: (0, 0)),
            pl.BlockSpec((C,), lambda b: (0,)),
            pl.BlockSpec((C, 2 * C), lambda b: (0, 0)),
            pl.BlockSpec((2 * C,), lambda b: (0,)),
        ],
        out_specs=pl.BlockSpec((1, S, 2 * C), lambda b: (b, 0, 0)),
        out_shape=jax.ShapeDtypeStruct((B, S, 2 * C), F32),
    )(g, new_xyz, w0t, b0, w1t, b1, w2t, b2)


def kernel(xyz, points, W0, b0, W1, b1, W2, b2):
    xt = jnp.transpose(xyz, (2, 0, 1))
    nxt = _fps(xt)
    new_xyz = jnp.transpose(nxt, (1, 0, 2))
    w0t = W0.T
    w1t = W1.T
    w2t = W2.T
    g0 = _g0(xyz, points, w0t)
    idx = _select(xyz, jnp.transpose(nxt, (1, 2, 0)))
    g = _sc_gather(g0.reshape(B * N, C),
                   idx.reshape(NW, NCHUNK, CHUNK))
    out = _mlp(g.reshape(B, K, S, C), new_xyz,
               w0t, b0, w1t, b1, w2t, b2)
    return (new_xyz, out)


# speedup vs baseline: 2.5123x; 1.2268x over previous
import functools

import jax
import jax.numpy as jnp
from jax import lax
from jax.experimental import pallas as pl
from jax.experimental.pallas import tpu as pltpu
from jax.experimental.pallas import tpu_sc as plsc

B, N, S, K, C = 16, 4096, 512, 32, 64
QC = 128
CH = 256
NCH = N // CH
F32 = jnp.float32
I32 = jnp.int32

NC_SC, NS_SC = 2, 16
NW = NC_SC * NS_SC
ROWS = B * K * S
ROWS_W = ROWS // NW
CHUNK = 128
NCHUNK = ROWS_W // CHUNK

_HI = jax.lax.Precision.HIGHEST


def _dot3(a, w):
    ah = a.astype(jnp.bfloat16)
    al = (a - ah.astype(F32)).astype(jnp.bfloat16)
    wh = w.astype(jnp.bfloat16)
    wl = (w - wh.astype(F32)).astype(jnp.bfloat16)

    def d(u, v):
        return jnp.dot(u, v, preferred_element_type=F32)

    return d(ah, wh) + (d(al, wh) + d(ah, wl))


def _fps_body(xt_ref, out_ref, dist_ref, far_ref):
    i = pl.program_id(0)

    @pl.when(i == 0)
    def _init():
        dist_ref[...] = jnp.full((B, N), 1e10, F32)
        far_ref[...] = jnp.zeros((B, 128), I32)

    x = xt_ref[0]
    y = xt_ref[1]
    z = xt_ref[2]
    far = far_ref[:, 0:1]
    lane = lax.broadcasted_iota(I32, (B, N), 1)
    oh = lane == far
    ninf = jnp.float32(-jnp.inf)
    dist0 = dist_ref[...]
    stack = jnp.concatenate(
        [jnp.where(oh, x, ninf), jnp.where(oh, y, ninf),
         jnp.where(oh, z, ninf)], axis=0)
    red = jnp.max(stack, axis=1, keepdims=True)
    cx = red[0:B]
    cy = red[B:2 * B]
    cz = red[2 * B:]
    out_ref[...] = jnp.concatenate([cx, cy, cz], axis=1)[None]
    dx = x - cx
    dy = y - cy
    dz = z - cz
    d = dx * dx + dy * dy + dz * dz
    dist = jnp.where(d < dist0, d, dist0)
    dist_ref[...] = dist
    m = jnp.max(dist, axis=1, keepdims=True)
    nxt = jnp.min(jnp.where(dist == m, lane, jnp.int32(N)),
                  axis=1, keepdims=True)
    far_ref[...] = jnp.broadcast_to(nxt, (B, 128))


def _fps(xt):
    return pl.pallas_call(
        _fps_body,
        grid=(S,),
        in_specs=[pl.BlockSpec((3, B, N), lambda i: (0, 0, 0))],
        out_specs=pl.BlockSpec((1, B, 3), lambda i: (i, 0, 0)),
        out_shape=jax.ShapeDtypeStruct((S, B, 3), F32),
        scratch_shapes=[pltpu.VMEM((B, N), F32), pltpu.VMEM((B, 128), I32)],
    )(xt)


def _g0_body(xyz_ref, pts_ref, w0t_ref, g0_ref):
    xyz = xyz_ref[0]
    pts = pts_ref[0]
    w = w0t_ref[...]
    g = _dot3(xyz, w[0:3]) + _dot3(pts, w[3:])
    g0_ref[...] = g[None]


def _g0(xyz, points, w0t):
    return pl.pallas_call(
        _g0_body,
        grid=(B,),
        in_specs=[
            pl.BlockSpec((1, N, 3), lambda b: (b, 0, 0)),
            pl.BlockSpec((1, N, C), lambda b: (b, 0, 0)),
            pl.BlockSpec((C + 3, C), lambda b: (0, 0)),
        ],
        out_specs=pl.BlockSpec((1, N, C), lambda b: (b, 0, 0)),
        out_shape=jax.ShapeDtypeStruct((B, N, C), F32),
    )(xyz, points, w0t)


def _sel_body(xyz_ref, nxt_ref, idx_ref, scr_ref):
    b = pl.program_id(0)
    q3 = nxt_ref[0]
    xq = q3[0:1, :]
    yq = q3[1:2, :]
    zq = q3[2:3, :]
    qsum = xq * xq + yq * yq + zq * zq
    q3b = q3.astype(jnp.bfloat16)
    for c in range(NCH):
        p = xyz_ref[0, pl.ds(c * CH, CH), :]
        psum = (p[:, 0:1] * p[:, 0:1] + p[:, 1:2] * p[:, 1:2]
                + p[:, 2:3] * p[:, 2:3])
        mm = jnp.dot(p.astype(jnp.bfloat16), q3b,
                     preferred_element_type=F32)
        d = -2.0 * mm
        d = d + qsum
        d = d + psum
        bits = lax.bitcast_convert_type(d, I32)
        sub = lax.broadcasted_iota(I32, (CH, QC), 0) + c * CH
        packed = (bits & jnp.int32(-4096)) | sub
        scr_ref[pl.ds(c * (CH // 128), CH // 128), :, :] = (
            packed.reshape(CH // 128, 128, QC))

    def _ce(v, d, asc):
        g = 32 // (2 * d)
        r = v.reshape((g, 2, d) + v.shape[1:])
        lo, hi = r[:, 0], r[:, 1]
        mn = jnp.minimum(lo, hi)
        mx = jnp.maximum(lo, hi)
        if isinstance(asc, bool):
            a, bb = (mn, mx) if asc else (mx, mn)
        else:
            a = jnp.where(asc, mn, mx)
            bb = jnp.where(asc, mx, mn)
        return jnp.concatenate([a[:, None], bb[:, None]], axis=1).reshape(v.shape)

    def _ce_pat(v, d, size):
        g = 32 // (2 * d)
        r = v.reshape((g, 2, d) + v.shape[1:])
        lo, hi = r[:, 0], r[:, 1]
        mn = jnp.minimum(lo, hi)
        mx = jnp.maximum(lo, hi)
        outs = []
        for gi in range(g):
            if ((gi * 2 * d) & size) == 0:
                outs.extend([mn[gi], mx[gi]])
            else:
                outs.extend([mx[gi], mn[gi]])
        return jnp.concatenate(outs, axis=0).reshape(v.shape)

    for j0 in range(0, 128, 8):
        v = scr_ref[:, pl.ds(j0, 8), :]
        for size in (2, 4, 8, 16):
            d = size // 2
            while d:
                v = _ce_pat(v, d, size)
                d //= 2
        asc = (j0 & 64) == 0
        for d in (16, 8, 4, 2, 1):
            v = _ce(v, d, asc)
        scr_ref[:, pl.ds(j0, 8), :] = v

    for h in (64, 32, 16, 8):
        step = min(8, h)
        for j0 in range(0, h, step):
            a = scr_ref[:, pl.ds(j0, step), :]
            bb = scr_ref[:, pl.ds(j0 + h, step), :]
            v = jnp.minimum(a, bb)
            if h // 2 >= 8:
                asc = (j0 & (h // 2)) == 0
            else:
                asc = (lax.broadcasted_iota(I32, (1, step, 1), 1)
                       & (h // 2)) == 0
            for d in (16, 8, 4, 2, 1):
                v = _ce(v, d, asc)
            scr_ref[:, pl.ds(j0, step), :] = v

    v = scr_ref[:, 0:8, :]
    for h in (4, 2, 1):
        c = jnp.minimum(v[:, :h], v[:, h:2 * h])
        if h > 1:
            asc = (lax.broadcasted_iota(I32, (1, h, 1), 1) & (h // 2)) == 0
        else:
            asc = True
        for d in (16, 8, 4, 2, 1):
            c = _ce(c, d, asc)
        v = c

    idx_ref[0] = (v[:, 0, :] & 4095) + b * N


def _select(xyz, nxt):
    return pl.pallas_call(
        _sel_body,
        grid=(B, S // QC),
        in_specs=[
            pl.BlockSpec((1, N, 3), lambda b, q: (b, 0, 0)),
            pl.BlockSpec((1, 3, QC), lambda b, q: (b, 0, q)),
        ],
        out_specs=pl.BlockSpec((1, K, QC), lambda b, q: (b, 0, q)),
        out_shape=jax.ShapeDtypeStruct((B, K, S), I32),
        scratch_shapes=[pltpu.VMEM((K, N // K, QC), I32)],
    )(xyz, nxt)


def _sc_gather(table, idx3):
    mesh = plsc.VectorSubcoreMesh(core_axis_name="c", subcore_axis_name="s")

    @functools.partial(
        pl.kernel,
        mesh=mesh,
        compiler_params=pltpu.CompilerParams(use_tc_tiling_on_sc=False),
        out_type=jax.ShapeDtypeStruct((ROWS, C), F32),
        scratch_types=[
            pltpu.VMEM((NCHUNK, CHUNK), I32),
            pltpu.VMEM((CHUNK, C), F32),
            pltpu.VMEM((CHUNK, C), F32),
            pltpu.SemaphoreType.DMA,
            pltpu.SemaphoreType.DMA,
        ],
    )
    def k(table_hbm, idx_hbm, out_hbm, idx_v, buf0, buf1, sem0, sem1):
        wid = lax.axis_index("s") * NC_SC + lax.axis_index("c")
        base = wid * ROWS_W
        pltpu.sync_copy(idx_hbm.at[wid], idx_v)

        def gather(c, buf, sem):
            pltpu.make_async_copy(table_hbm.at[idx_v.at[c]], buf, sem).start()

        def drain(buf, sem):
            pltpu.make_async_copy(table_hbm.at[idx_v.at[0]], buf, sem).wait()

        gather(0, buf0, sem0)
        gather(1, buf1, sem1)

        def body(c, carry):
            drain(buf0, sem0)
            pltpu.sync_copy(buf0, out_hbm.at[pl.ds(base + 2 * c * CHUNK, CHUNK)])

            @pl.when(c < NCHUNK // 2 - 1)
            def _():
                gather(2 * c + 2, buf0, sem0)

            drain(buf1, sem1)
            pltpu.sync_copy(
                buf1, out_hbm.at[pl.ds(base + (2 * c + 1) * CHUNK, CHUNK)])

            @pl.when(c < NCHUNK // 2 - 1)
            def _():
                gather(2 * c + 3, buf1, sem1)

            return carry

        lax.fori_loop(0, NCHUNK // 2, body, 0)

    return k(table, idx3)


def _mlp_body(g_ref, nxy_ref, w0t_ref, b0_ref, w1t_ref, b1_ref,
              w2t_ref, b2_ref, out_ref):
    nxy = nxy_ref[0]
    q0 = (b0_ref[...][None, :]
          - jnp.dot(nxy, w0t_ref[0:3], precision=_HI,
                    preferred_element_type=F32))
    b1 = b1_ref[...][None, :]
    b2 = b2_ref[...][None, :]
    w1 = w1t_ref[...]
    w2 = w2t_ref[...]
    a = jnp.maximum(g_ref[0] + q0[None], 0.0).reshape(K * S, C)
    h = jnp.maximum(
        _dot3(a, w1) + b1, 0.0)
    o = jnp.maximum(
        _dot3(h, w2) + b2, 0.0)
    out_ref[...] = jnp.max(o.reshape(K, S, 2 * C), axis=0)[None]


def _mlp(g, new_xyz, w0t, b0, w1t, b1, w2t, b2):
    return pl.pallas_call(
        _mlp_body,
        grid=(B,),
        in_specs=[
            pl.BlockSpec((1, K, S, C), lambda b: (b, 0, 0, 0)),
            pl.BlockSpec((1, S, 3), lambda b: (b, 0, 0)),
            pl.BlockSpec((C + 3, C), lambda b: (0, 0)),
            pl.BlockSpec((C,), lambda b: (0,)),
            pl.BlockSpec((C, C), lambda b: (0, 0)),
            pl.BlockSpec((C,), lambda b: (0,)),
            pl.BlockSpec((C, 2 * C), lambda b: (0, 0)),
            pl.BlockSpec((2 * C,), lambda b: (0,)),
        ],
        out_specs=pl.BlockSpec((1, S, 2 * C), lambda b: (b, 0, 0)),
        out_shape=jax.ShapeDtypeStruct((B, S, 2 * C), F32),
    )(g, new_xyz, w0t, b0, w1t, b1, w2t, b2)


def kernel(xyz, points, W0, b0, W1, b1, W2, b2):
    xt = jnp.transpose(xyz, (2, 0, 1))
    nxt = _fps(xt)
    new_xyz = jnp.transpose(nxt, (1, 0, 2))
    w0t = W0.T
    w1t = W1.T
    w2t = W2.T
    g0 = _g0(xyz, points, w0t)
    idx = _select(xyz, jnp.transpose(nxt, (1, 2, 0)))
    g = _sc_gather(g0.reshape(B * N, C),
                   idx.reshape(NW, NCHUNK, CHUNK))
    out = _mlp(g.reshape(B, K, S, C), new_xyz,
               w0t, b0, w1t, b1, w2t, b2)
    return (new_xyz, out)


# speedup vs baseline: 2.6327x; 1.0479x over previous
import functools

import jax
import jax.numpy as jnp
from jax import lax
from jax.experimental import pallas as pl
from jax.experimental.pallas import tpu as pltpu
from jax.experimental.pallas import tpu_sc as plsc

B, N, S, K, C = 16, 4096, 512, 32, 64
QC = 256
CH = 256
NCH = N // CH
F32 = jnp.float32
I32 = jnp.int32

NC_SC, NS_SC = 2, 16
NW = NC_SC * NS_SC
ROWS = B * K * S
ROWS_W = ROWS // NW
CHUNK = 128
NCHUNK = ROWS_W // CHUNK

_HI = jax.lax.Precision.HIGHEST


def _dot3(a, w):
    ah = a.astype(jnp.bfloat16)
    al = (a - ah.astype(F32)).astype(jnp.bfloat16)
    wh = w.astype(jnp.bfloat16)
    wl = (w - wh.astype(F32)).astype(jnp.bfloat16)

    def d(u, v):
        return jnp.dot(u, v, preferred_element_type=F32)

    return d(ah, wh) + (d(al, wh) + d(ah, wl))


def _fps_body(xt_ref, out_ref, dist_ref, far_ref):
    i = pl.program_id(0)

    @pl.when(i == 0)
    def _init():
        dist_ref[...] = jnp.full((B, N), 1e10, F32)
        far_ref[...] = jnp.zeros((B, 128), I32)

    x = xt_ref[0]
    y = xt_ref[1]
    z = xt_ref[2]
    far = far_ref[:, 0:1]
    lane = lax.broadcasted_iota(I32, (B, N), 1)
    oh = lane == far
    ninf = jnp.float32(-jnp.inf)
    dist0 = dist_ref[...]
    stack = jnp.concatenate(
        [jnp.where(oh, x, ninf), jnp.where(oh, y, ninf),
         jnp.where(oh, z, ninf)], axis=0)
    red = jnp.max(stack, axis=1, keepdims=True)
    cx = red[0:B]
    cy = red[B:2 * B]
    cz = red[2 * B:]
    out_ref[...] = jnp.concatenate([cx, cy, cz], axis=1)[None]
    dx = x - cx
    dy = y - cy
    dz = z - cz
    d = dx * dx + dy * dy + dz * dz
    dist = jnp.where(d < dist0, d, dist0)
    dist_ref[...] = dist
    m = jnp.max(dist, axis=1, keepdims=True)
    nxt = jnp.min(jnp.where(dist == m, lane, jnp.int32(N)),
                  axis=1, keepdims=True)
    far_ref[...] = jnp.broadcast_to(nxt, (B, 128))


def _fps(xt):
    return pl.pallas_call(
        _fps_body,
        grid=(S,),
        in_specs=[pl.BlockSpec((3, B, N), lambda i: (0, 0, 0))],
        out_specs=pl.BlockSpec((1, B, 3), lambda i: (i, 0, 0)),
        out_shape=jax.ShapeDtypeStruct((S, B, 3), F32),
        scratch_shapes=[pltpu.VMEM((B, N), F32), pltpu.VMEM((B, 128), I32)],
    )(xt)


def _g0_body(xyz_ref, pts_ref, w0t_ref, g0_ref):
    xyz = xyz_ref[0]
    pts = pts_ref[0]
    w = w0t_ref[...]
    g = _dot3(xyz, w[0:3]) + _dot3(pts, w[3:])
    g0_ref[...] = g[None]


def _g0(xyz, points, w0t):
    return pl.pallas_call(
        _g0_body,
        grid=(B,),
        in_specs=[
            pl.BlockSpec((1, N, 3), lambda b: (b, 0, 0)),
            pl.BlockSpec((1, N, C), lambda b: (b, 0, 0)),
            pl.BlockSpec((C + 3, C), lambda b: (0, 0)),
        ],
        out_specs=pl.BlockSpec((1, N, C), lambda b: (b, 0, 0)),
        out_shape=jax.ShapeDtypeStruct((B, N, C), F32),
    )(xyz, points, w0t)


def _sel_body(xyz_ref, nxt_ref, idx_ref, scr_ref):
    b = pl.program_id(0)
    q3 = nxt_ref[0]
    xq = q3[0:1, :]
    yq = q3[1:2, :]
    zq = q3[2:3, :]
    qsum = xq * xq + yq * yq + zq * zq
    q3b = q3.astype(jnp.bfloat16)
    for c in range(NCH):
        p = xyz_ref[0, pl.ds(c * CH, CH), :]
        psum = (p[:, 0:1] * p[:, 0:1] + p[:, 1:2] * p[:, 1:2]
                + p[:, 2:3] * p[:, 2:3])
        mm = jnp.dot(p.astype(jnp.bfloat16), q3b,
                     preferred_element_type=F32)
        d = -2.0 * mm
        d = d + qsum
        d = d + psum
        bits = lax.bitcast_convert_type(d, I32)
        sub = lax.broadcasted_iota(I32, (CH, QC), 0) + c * CH
        packed = (bits & jnp.int32(-4096)) | sub
        scr_ref[pl.ds(c * (CH // 128), CH // 128), :, :] = (
            packed.reshape(CH // 128, 128, QC))

    def _ce(v, d, asc):
        g = 32 // (2 * d)
        r = v.reshape((g, 2, d) + v.shape[1:])
        lo, hi = r[:, 0], r[:, 1]
        mn = jnp.minimum(lo, hi)
        mx = jnp.maximum(lo, hi)
        if isinstance(asc, bool):
            a, bb = (mn, mx) if asc else (mx, mn)
        else:
            a = jnp.where(asc, mn, mx)
            bb = jnp.where(asc, mx, mn)
        return jnp.concatenate([a[:, None], bb[:, None]], axis=1).reshape(v.shape)

    def _ce_pat(v, d, size):
        g = 32 // (2 * d)
        r = v.reshape((g, 2, d) + v.shape[1:])
        lo, hi = r[:, 0], r[:, 1]
        mn = jnp.minimum(lo, hi)
        mx = jnp.maximum(lo, hi)
        outs = []
        for gi in range(g):
            if ((gi * 2 * d) & size) == 0:
                outs.extend([mn[gi], mx[gi]])
            else:
                outs.extend([mx[gi], mn[gi]])
        return jnp.concatenate(outs, axis=0).reshape(v.shape)

    for j0 in range(0, 128, 8):
        v = scr_ref[:, pl.ds(j0, 8), :]
        for size in (2, 4, 8, 16):
            d = size // 2
            while d:
                v = _ce_pat(v, d, size)
                d //= 2
        asc = (j0 & 64) == 0
        for d in (16, 8, 4, 2, 1):
            v = _ce(v, d, asc)
        scr_ref[:, pl.ds(j0, 8), :] = v

    for h in (64, 32, 16, 8):
        step = min(8, h)
        for j0 in range(0, h, step):
            a = scr_ref[:, pl.ds(j0, step), :]
            bb = scr_ref[:, pl.ds(j0 + h, step), :]
            v = jnp.minimum(a, bb)
            if h // 2 >= 8:
                asc = (j0 & (h // 2)) == 0
            else:
                asc = (lax.broadcasted_iota(I32, (1, step, 1), 1)
                       & (h // 2)) == 0
            for d in (16, 8, 4, 2, 1):
                v = _ce(v, d, asc)
            scr_ref[:, pl.ds(j0, step), :] = v

    v = scr_ref[:, 0:8, :]
    for h in (4, 2, 1):
        c = jnp.minimum(v[:, :h], v[:, h:2 * h])
        if h > 1:
            asc = (lax.broadcasted_iota(I32, (1, h, 1), 1) & (h // 2)) == 0
        else:
            asc = True
        for d in (16, 8, 4, 2, 1):
            c = _ce(c, d, asc)
        v = c

    idx_ref[0] = (v[:, 0, :] & 4095) + b * N


def _select(xyz, nxt):
    return pl.pallas_call(
        _sel_body,
        grid=(B, S // QC),
        in_specs=[
            pl.BlockSpec((1, N, 3), lambda b, q: (b, 0, 0)),
            pl.BlockSpec((1, 3, QC), lambda b, q: (b, 0, q)),
        ],
        out_specs=pl.BlockSpec((1, K, QC), lambda b, q: (b, 0, q)),
        out_shape=jax.ShapeDtypeStruct((B, K, S), I32),
        scratch_shapes=[pltpu.VMEM((K, N // K, QC), I32)],
    )(xyz, nxt)


def _sc_gather(table, idx3):
    mesh = plsc.VectorSubcoreMesh(core_axis_name="c", subcore_axis_name="s")

    @functools.partial(
        pl.kernel,
        mesh=mesh,
        compiler_params=pltpu.CompilerParams(use_tc_tiling_on_sc=False),
        out_type=jax.ShapeDtypeStruct((ROWS, C), F32),
        scratch_types=[
            pltpu.VMEM((NCHUNK, CHUNK), I32),
            pltpu.VMEM((CHUNK, C), F32),
            pltpu.VMEM((CHUNK, C), F32),
            pltpu.SemaphoreType.DMA,
            pltpu.SemaphoreType.DMA,
        ],
    )
    def k(table_hbm, idx_hbm, out_hbm, idx_v, buf0, buf1, sem0, sem1):
        wid = lax.axis_index("s") * NC_SC + lax.axis_index("c")
        base = wid * ROWS_W
        pltpu.sync_copy(idx_hbm.at[wid], idx_v)

        def gather(c, buf, sem):
            pltpu.make_async_copy(table_hbm.at[idx_v.at[c]], buf, sem).start()

        def drain(buf, sem):
            pltpu.make_async_copy(table_hbm.at[idx_v.at[0]], buf, sem).wait()

        gather(0, buf0, sem0)
        gather(1, buf1, sem1)

        def body(c, carry):
            drain(buf0, sem0)
            pltpu.sync_copy(buf0, out_hbm.at[pl.ds(base + 2 * c * CHUNK, CHUNK)])

            @pl.when(c < NCHUNK // 2 - 1)
            def _():
                gather(2 * c + 2, buf0, sem0)

            drain(buf1, sem1)
            pltpu.sync_copy(
                buf1, out_hbm.at[pl.ds(base + (2 * c + 1) * CHUNK, CHUNK)])

            @pl.when(c < NCHUNK // 2 - 1)
            def _():
                gather(2 * c + 3, buf1, sem1)

            return carry

        lax.fori_loop(0, NCHUNK // 2, body, 0)

    return k(table, idx3)


def _mlp_body(g_ref, nxy_ref, w0t_ref, b0_ref, w1t_ref, b1_ref,
              w2t_ref, b2_ref, out_ref):
    nxy = nxy_ref[0]
    q0 = (b0_ref[...][None, :]
          - jnp.dot(nxy, w0t_ref[0:3], precision=_HI,
                    preferred_element_type=F32))
    b1 = b1_ref[...][None, :]
    b2 = b2_ref[...][None, :]
    w1 = w1t_ref[...]
    w2 = w2t_ref[...]
    a = jnp.maximum(g_ref[0] + q0[None], 0.0).reshape(K * S, C)
    h = jnp.maximum(
        _dot3(a, w1) + b1, 0.0)
    o = jnp.maximum(
        _dot3(h, w2) + b2, 0.0)
    out_ref[...] = jnp.max(o.reshape(K, S, 2 * C), axis=0)[None]


def _mlp(g, new_xyz, w0t, b0, w1t, b1, w2t, b2):
    return pl.pallas_call(
        _mlp_body,
        grid=(B,),
        in_specs=[
            pl.BlockSpec((1, K, S, C), lambda b: (b, 0, 0, 0)),
            pl.BlockSpec((1, S, 3), lambda b: (b, 0, 0)),
            pl.BlockSpec((C + 3, C), lambda b: (0, 0)),
            pl.BlockSpec((C,), lambda b: (0,)),
            pl.BlockSpec((C, C), lambda b: (0, 0)),
            pl.BlockSpec((C,), lambda b: (0,)),
            pl.BlockSpec((C, 2 * C), lambda b: (0, 0)),
            pl.BlockSpec((2 * C,), lambda b: (0,)),
        ],
        out_specs=pl.BlockSpec((1, S, 2 * C), lambda b: (b, 0, 0)),
        out_shape=jax.ShapeDtypeStruct((B, S, 2 * C), F32),
    )(g, new_xyz, w0t, b0, w1t, b1, w2t, b2)


def kernel(xyz, points, W0, b0, W1, b1, W2, b2):
    xt = jnp.transpose(xyz, (2, 0, 1))
    nxt = _fps(xt)
    new_xyz = jnp.transpose(nxt, (1, 0, 2))
    w0t = W0.T
    w1t = W1.T
    w2t = W2.T
    g0 = _g0(xyz, points, w0t)
    idx = _select(xyz, jnp.transpose(nxt, (1, 2, 0)))
    g = _sc_gather(g0.reshape(B * N, C),
                   idx.reshape(NW, NCHUNK, CHUNK))
    out = _mlp(g.reshape(B, K, S, C), new_xyz,
               w0t, b0, w1t, b1, w2t, b2)
    return (new_xyz, out)
